# Initial kernel scaffold; baseline (speedup 1.0000x reference)
import jax
import jax.numpy as jnp
from jax.experimental import pallas as pl


def kernel(feat_matrix, adj_matrix, get_item_index, set_index, val_index, mask_matrix, W1, b1, W2, b2, W_out, b_out):
    raise NotImplementedError("write your pallas kernel here")


# speedup vs baseline: 2221.5551x; 2221.5551x over previous
import functools

import jax
import jax.numpy as jnp
from jax.experimental import pallas as pl

N = 2048
D_IN = 128
OUT = 64
BLK = 256
NBLK = N // BLK


def _pass1_body(a_ref, x_ref, dis_ref, y_ref):
    j = pl.program_id(0)
    a = a_ref[...]
    rowsum = jnp.sum(a.astype(jnp.float32), axis=1, keepdims=True)
    col = jax.lax.broadcasted_iota(jnp.int32, (BLK, N), 1)
    row_g = jax.lax.broadcasted_iota(jnp.int32, (BLK, N), 0) + j * BLK
    diag = jnp.sum(jnp.where(col == row_g, a, 0).astype(jnp.float32),
                   axis=1, keepdims=True)
    deg = rowsum - diag
    dis = jnp.where(deg > 0, jax.lax.rsqrt(deg), 0.0)
    dis_ref[...] = jnp.broadcast_to(dis, (BLK, D_IN))
    y_ref[...] = dis * x_ref[...]


def _pass2_body(a_ref, y_ref, dis_ref, x_ref, w1_ref, w20_ref, w21_ref,
                wt_ref, wb_ref, b1_ref, b2_ref, bo_ref, out_ref):
    j = pl.program_id(0)
    a = a_ref[...]
    row = jax.lax.broadcasted_iota(jnp.int32, (N, BLK), 0)
    col_g = jax.lax.broadcasted_iota(jnp.int32, (N, BLK), 1) + j * BLK
    af = jnp.where(row == col_g, 0, a).astype(jnp.float32)
    t = jax.lax.dot_general(af, y_ref[...], (((0,), (0,)), ((), ())),
                            preferred_element_type=jnp.float32)
    tx1 = -dis_ref[...] * t
    x = x_ref[...]
    y1 = jnp.dot(x, w1_ref[...], preferred_element_type=jnp.float32) + b1_ref[...]
    y2 = (jnp.dot(x, w20_ref[...], preferred_element_type=jnp.float32)
          + jnp.dot(tx1, w21_ref[...], preferred_element_type=jnp.float32)
          + b2_ref[...])
    z = (jnp.dot(y1, wt_ref[...], preferred_element_type=jnp.float32)
         + jnp.dot(y2, wb_ref[...], preferred_element_type=jnp.float32)
         + bo_ref[...])
    m = jnp.max(z, axis=-1, keepdims=True)
    e = jnp.exp(z - m)
    out_ref[...] = e / jnp.sum(e, axis=-1, keepdims=True)


@functools.partial(jax.jit, static_argnames=("interpret",))
def _run(adj, x, w1, w20, w21, wt, wb, b1, b2, bo, interpret=False):
    dis, y = pl.pallas_call(
        _pass1_body,
        grid=(NBLK,),
        in_specs=[
            pl.BlockSpec((BLK, N), lambda j: (j, 0)),
            pl.BlockSpec((BLK, D_IN), lambda j: (j, 0)),
        ],
        out_specs=[
            pl.BlockSpec((BLK, D_IN), lambda j: (j, 0)),
            pl.BlockSpec((BLK, D_IN), lambda j: (j, 0)),
        ],
        out_shape=[
            jax.ShapeDtypeStruct((N, D_IN), jnp.float32),
            jax.ShapeDtypeStruct((N, D_IN), jnp.float32),
        ],
        interpret=interpret,
    )(adj, x)

    out = pl.pallas_call(
        _pass2_body,
        grid=(NBLK,),
        in_specs=[
            pl.BlockSpec((N, BLK), lambda j: (0, j)),
            pl.BlockSpec((N, D_IN), lambda j: (0, 0)),
            pl.BlockSpec((BLK, D_IN), lambda j: (j, 0)),
            pl.BlockSpec((BLK, D_IN), lambda j: (j, 0)),
            pl.BlockSpec((D_IN, D_IN), lambda j: (0, 0)),
            pl.BlockSpec((D_IN, D_IN), lambda j: (0, 0)),
            pl.BlockSpec((D_IN, D_IN), lambda j: (0, 0)),
            pl.BlockSpec((D_IN, OUT), lambda j: (0, 0)),
            pl.BlockSpec((D_IN, OUT), lambda j: (0, 0)),
            pl.BlockSpec((1, D_IN), lambda j: (0, 0)),
            pl.BlockSpec((1, D_IN), lambda j: (0, 0)),
            pl.BlockSpec((1, OUT), lambda j: (0, 0)),
        ],
        out_specs=pl.BlockSpec((BLK, OUT), lambda j: (j, 0)),
        out_shape=jax.ShapeDtypeStruct((N, OUT), jnp.float32),
        interpret=interpret,
    )(adj, y, dis, x, w1, w20, w21, wt, wb, b1, b2, bo)
    return out


def kernel(feat_matrix, adj_matrix, get_item_index, set_index, val_index,
           mask_matrix, W1, b1, W2, b2, W_out, b_out, interpret=False):
    adj = adj_matrix[:, :, 0]
    return _run(adj, feat_matrix,
                W1[0], W2[0], W2[1],
                W_out[:D_IN], W_out[D_IN:],
                b1.reshape(1, D_IN), b2.reshape(1, D_IN),
                b_out.reshape(1, OUT), interpret=interpret)


# speedup vs baseline: 2717.7820x; 1.2234x over previous
import functools

import jax
import jax.numpy as jnp
from jax.experimental import pallas as pl
from jax.experimental.pallas import tpu as pltpu

N = 2048
D_IN = 128
OUT = 64
BLK = 256
NBLK = N // BLK


def _body(a_ref, x_ref, w1_ref, w20_ref, w21_ref, wt_ref, wb_ref,
          b1_ref, b2_ref, bo_ref, out_ref, t_ref, dis_ref):
    j = pl.program_id(0)

    @pl.when(j < NBLK)
    def _phase1():
        a = a_ref[...]
        col = jax.lax.broadcasted_iota(jnp.int32, (BLK, N), 1)
        row_g = jax.lax.broadcasted_iota(jnp.int32, (BLK, N), 0) + j * BLK
        af = jnp.where(col == row_g, 0, a).astype(jnp.float32)
        deg = jnp.sum(af, axis=1, keepdims=True)
        dis = jnp.where(deg > 0, jax.lax.rsqrt(deg), 0.0)
        dis_ref[pl.ds(j * BLK, BLK), :] = jnp.broadcast_to(dis, (BLK, D_IN))
        z = dis * x_ref[pl.ds(j * BLK, BLK), :]
        contrib = jax.lax.dot_general(af, z, (((0,), (0,)), ((), ())),
                                      preferred_element_type=jnp.float32)

        @pl.when(j == 0)
        def _init():
            t_ref[...] = contrib

        @pl.when(j > 0)
        def _acc():
            t_ref[...] += contrib

    @pl.when(j == NBLK)
    def _phase2():
        tx1 = -dis_ref[...] * t_ref[...]
        x = x_ref[...]
        y1 = jnp.dot(x, w1_ref[...], preferred_element_type=jnp.float32) + b1_ref[...]
        y2 = (jnp.dot(x, w20_ref[...], preferred_element_type=jnp.float32)
              + jnp.dot(tx1, w21_ref[...], preferred_element_type=jnp.float32)
              + b2_ref[...])
        z = (jnp.dot(y1, wt_ref[...], preferred_element_type=jnp.float32)
             + jnp.dot(y2, wb_ref[...], preferred_element_type=jnp.float32)
             + bo_ref[...])
        m = jnp.max(z, axis=-1, keepdims=True)
        e = jnp.exp(z - m)
        out_ref[...] = e / jnp.sum(e, axis=-1, keepdims=True)


@functools.partial(jax.jit, static_argnames=("interpret",))
def _run(adj, x, w1, w20, w21, wt, wb, b1, b2, bo, interpret=False):
    const = lambda j: (0, 0)
    out = pl.pallas_call(
        _body,
        grid=(NBLK + 1,),
        in_specs=[
            pl.BlockSpec((BLK, N), lambda j: (jnp.minimum(j, NBLK - 1), 0)),
            pl.BlockSpec((N, D_IN), const),
            pl.BlockSpec((D_IN, D_IN), const),
            pl.BlockSpec((D_IN, D_IN), const),
            pl.BlockSpec((D_IN, D_IN), const),
            pl.BlockSpec((D_IN, OUT), const),
            pl.BlockSpec((D_IN, OUT), const),
            pl.BlockSpec((1, D_IN), const),
            pl.BlockSpec((1, D_IN), const),
            pl.BlockSpec((1, OUT), const),
        ],
        out_specs=pl.BlockSpec((N, OUT), const),
        out_shape=jax.ShapeDtypeStruct((N, OUT), jnp.float32),
        scratch_shapes=[
            pltpu.VMEM((N, D_IN), jnp.float32),
            pltpu.VMEM((N, D_IN), jnp.float32),
        ],
        interpret=interpret,
    )(adj, x, w1, w20, w21, wt, wb, b1, b2, bo)
    return out


def kernel(feat_matrix, adj_matrix, get_item_index, set_index, val_index,
           mask_matrix, W1, b1, W2, b2, W_out, b_out, interpret=False):
    adj = adj_matrix[:, :, 0]
    return _run(adj, feat_matrix,
                W1[0], W2[0], W2[1],
                W_out[:D_IN], W_out[D_IN:],
                b1.reshape(1, D_IN), b2.reshape(1, D_IN),
                b_out.reshape(1, OUT), interpret=interpret)
